# baseline (speedup 1.0000x reference)
import functools

import jax
import jax.numpy as jnp
from jax import lax
from jax.experimental import pallas as pl
from jax.experimental.pallas import tpu as pltpu
from jax.experimental.pallas import tpu_sc as plsc

B, C, H, W = 8, 3, 512, 512
HW = H * W
NPIX = B * HW
NWORKERS = 32
PIX_PER_WORKER = NPIX // NWORKERS
CHUNK = 8192
NCHUNK = PIX_PER_WORKER // CHUNK
NVEC = CHUNK // 16


def _body(img_hbm, w_hbm, k_hbm, out_hbm,
          rbuf, gbuf, bbuf, idxb, wbuf, kbuf, outr, outg, outb,
          sem_w, sem_k):
    cid = lax.axis_index("c")
    sid = lax.axis_index("s")
    wid = sid * 2 + cid
    b = wid // 4
    off = (wid % 4) * PIX_PER_WORKER
    base_r = b * (3 * HW) + off
    base_g = base_r + HW
    base_b = base_r + 2 * HW

    def chunk_body(ci, _):
        o = ci * CHUNK
        pltpu.sync_copy(img_hbm.at[pl.ds(base_r + o, CHUNK)], rbuf)
        pltpu.sync_copy(img_hbm.at[pl.ds(base_g + o, CHUNK)], gbuf)
        pltpu.sync_copy(img_hbm.at[pl.ds(base_b + o, CHUNK)], bbuf)

        def idx_body(i, _):
            s = pl.ds(i * 16, 16)
            idxb[s] = (rbuf[s] << 16) | (gbuf[s] << 8) | bbuf[s]
            return _

        lax.fori_loop(0, NVEC, idx_body, None)

        cp_w = pltpu.async_copy(w_hbm.at[idxb], wbuf, sem_w)
        cp_k = pltpu.async_copy(k_hbm.at[idxb], kbuf, sem_k)
        cp_w.wait()
        cp_k.wait()

        def out_body(i, _):
            s = pl.ds(i * 16, 16)
            sc = wbuf[s]
            sh = kbuf[s]
            outr[s] = sc * rbuf[s].astype(jnp.float32) + sh
            outg[s] = sc * gbuf[s].astype(jnp.float32) + sh
            outb[s] = sc * bbuf[s].astype(jnp.float32) + sh
            return _

        lax.fori_loop(0, NVEC, out_body, None)

        pltpu.sync_copy(outr, out_hbm.at[pl.ds(base_r + o, CHUNK)])
        pltpu.sync_copy(outg, out_hbm.at[pl.ds(base_g + o, CHUNK)])
        pltpu.sync_copy(outb, out_hbm.at[pl.ds(base_b + o, CHUNK)])
        return _

    lax.fori_loop(0, NCHUNK, chunk_body, None)


@jax.jit
def _colormap_sc(img_flat, w_flat, k_flat):
    mesh = plsc.VectorSubcoreMesh(core_axis_name="c", subcore_axis_name="s")
    f = pl.kernel(
        _body,
        out_type=jax.ShapeDtypeStruct((B * 3 * HW,), jnp.float32),
        mesh=mesh,
        scratch_types=[
            pltpu.VMEM((CHUNK,), jnp.int32),
            pltpu.VMEM((CHUNK,), jnp.int32),
            pltpu.VMEM((CHUNK,), jnp.int32),
            pltpu.VMEM((CHUNK,), jnp.int32),
            pltpu.VMEM((CHUNK,), jnp.float32),
            pltpu.VMEM((CHUNK,), jnp.float32),
            pltpu.VMEM((CHUNK,), jnp.float32),
            pltpu.VMEM((CHUNK,), jnp.float32),
            pltpu.VMEM((CHUNK,), jnp.float32),
            pltpu.SemaphoreType.DMA,
            pltpu.SemaphoreType.DMA,
        ],
    )
    return f(img_flat, w_flat, k_flat)


def kernel(img, w, k):
    out_flat = _colormap_sc(img.reshape(-1), w.reshape(-1), k.reshape(-1))
    return out_flat.reshape(B, 3, H, W)


# speedup vs baseline: 1.1195x; 1.1195x over previous
import jax
import jax.numpy as jnp
from jax import lax
from jax.experimental import pallas as pl
from jax.experimental.pallas import tpu as pltpu
from jax.experimental.pallas import tpu_sc as plsc

B, C, H, W = 8, 3, 512, 512
HW = H * W
NPIX = B * HW
NWORKERS = 32
PIX_PER_WORKER = NPIX // NWORKERS
CHUNK = 4096
NCHUNK = PIX_PER_WORKER // CHUNK
NVEC = CHUNK // 16


def _body(img_hbm, w_hbm, k_hbm, out_hbm,
          rbuf, gbuf, bbuf, idxb, wbuf, kbuf, outr, outg, outb,
          sem_ld, sem_g, sem_st):
    cid = lax.axis_index("c")
    sid = lax.axis_index("s")
    wid = sid * 2 + cid
    b = wid // 4
    off = (wid % 4) * PIX_PER_WORKER
    base_r = b * (3 * HW) + off
    base_g = base_r + HW
    base_b = base_r + 2 * HW

    def issue_load(c):
        s = (c % 3) * CHUNK
        o = c * CHUNK
        return [
            pltpu.async_copy(img_hbm.at[pl.ds(base_r + o, CHUNK)],
                             rbuf.at[pl.ds(s, CHUNK)], sem_ld.at[c % 3]),
            pltpu.async_copy(img_hbm.at[pl.ds(base_g + o, CHUNK)],
                             gbuf.at[pl.ds(s, CHUNK)], sem_ld.at[c % 3]),
            pltpu.async_copy(img_hbm.at[pl.ds(base_b + o, CHUNK)],
                             bbuf.at[pl.ds(s, CHUNK)], sem_ld.at[c % 3]),
        ]

    def idx_loop(c):
        s = (c % 3) * CHUNK

        def body(i, _):
            sl = pl.ds(s + i * 16, 16)
            idxb[sl] = (rbuf[sl] << 16) | (gbuf[sl] << 8) | bbuf[sl]
            return _

        lax.fori_loop(0, NVEC, body, None)

    def issue_gather(c):
        s = (c % 2) * CHUNK
        src = pl.ds((c % 3) * CHUNK, CHUNK)
        return [
            pltpu.async_copy(w_hbm.at[idxb.at[src]],
                             wbuf.at[pl.ds(s, CHUNK)], sem_g.at[c % 2]),
            pltpu.async_copy(k_hbm.at[idxb.at[src]],
                             kbuf.at[pl.ds(s, CHUNK)], sem_g.at[c % 2]),
        ]

    def out_loop(c):
        s3 = (c % 3) * CHUNK
        s2 = (c % 2) * CHUNK

        def body(i, _):
            a = pl.ds(s3 + i * 16, 16)
            d = pl.ds(s2 + i * 16, 16)
            sc = wbuf[d]
            sh = kbuf[d]
            outr[d] = sc * rbuf[a].astype(jnp.float32) + sh
            outg[d] = sc * gbuf[a].astype(jnp.float32) + sh
            outb[d] = sc * bbuf[a].astype(jnp.float32) + sh
            return _

        lax.fori_loop(0, NVEC, body, None)

    def issue_store(c):
        s = (c % 2) * CHUNK
        o = c * CHUNK
        return [
            pltpu.async_copy(outr.at[pl.ds(s, CHUNK)],
                             out_hbm.at[pl.ds(base_r + o, CHUNK)],
                             sem_st.at[c % 2]),
            pltpu.async_copy(outg.at[pl.ds(s, CHUNK)],
                             out_hbm.at[pl.ds(base_g + o, CHUNK)],
                             sem_st.at[c % 2]),
            pltpu.async_copy(outb.at[pl.ds(s, CHUNK)],
                             out_hbm.at[pl.ds(base_b + o, CHUNK)],
                             sem_st.at[c % 2]),
        ]

    loads = {}
    gathers = {}
    stores = {}
    loads[0] = issue_load(0)

    for c in range(NCHUNK):
        for cp in loads.pop(c):
            cp.wait()
        idx_loop(c)
        gathers[c] = issue_gather(c)
        if c + 1 < NCHUNK:
            loads[c + 1] = issue_load(c + 1)
        if c > 0:
            for cp in gathers.pop(c - 1):
                cp.wait()
            if c - 3 in stores:
                for cp in stores.pop(c - 3):
                    cp.wait()
            out_loop(c - 1)
            stores[c - 1] = issue_store(c - 1)

    for cp in gathers.pop(NCHUNK - 1):
        cp.wait()
    if NCHUNK - 3 in stores:
        for cp in stores.pop(NCHUNK - 3):
            cp.wait()
    out_loop(NCHUNK - 1)
    stores[NCHUNK - 1] = issue_store(NCHUNK - 1)
    for c in sorted(stores):
        for cp in stores[c]:
            cp.wait()


@jax.jit
def _colormap_sc(img_flat, w_flat, k_flat):
    mesh = plsc.VectorSubcoreMesh(core_axis_name="c", subcore_axis_name="s")
    f = pl.kernel(
        _body,
        out_type=jax.ShapeDtypeStruct((B * 3 * HW,), jnp.float32),
        mesh=mesh,
        scratch_types=[
            pltpu.VMEM((3 * CHUNK,), jnp.int32),
            pltpu.VMEM((3 * CHUNK,), jnp.int32),
            pltpu.VMEM((3 * CHUNK,), jnp.int32),
            pltpu.VMEM((3 * CHUNK,), jnp.int32),
            pltpu.VMEM((2 * CHUNK,), jnp.float32),
            pltpu.VMEM((2 * CHUNK,), jnp.float32),
            pltpu.VMEM((2 * CHUNK,), jnp.float32),
            pltpu.VMEM((2 * CHUNK,), jnp.float32),
            pltpu.VMEM((2 * CHUNK,), jnp.float32),
            pltpu.SemaphoreType.DMA((3,)),
            pltpu.SemaphoreType.DMA((2,)),
            pltpu.SemaphoreType.DMA((2,)),
        ],
    )
    return f(img_flat, w_flat, k_flat)


def kernel(img, w, k):
    out_flat = _colormap_sc(img.reshape(-1), w.reshape(-1), k.reshape(-1))
    return out_flat.reshape(B, 3, H, W)
